# baseline (speedup 1.0000x reference)
import functools

import jax
import jax.numpy as jnp
from jax import lax
from jax.experimental import pallas as pl
from jax.experimental.pallas import tpu as pltpu
from jax.experimental.pallas import tpu_sc as plsc

B = 1024
NI = 1000
NCAND = 100
NSEEN = 200

NCORES = 2
NSUB = 16
NW = NCORES * NSUB
ROWS = B // NW

_mesh = plsc.VectorSubcoreMesh(core_axis_name="c", subcore_axis_name="s")
_sc_params = pltpu.CompilerParams(
    needs_layout_passes=False, use_tc_tiling_on_sc=False
)


def _wid():
    return lax.axis_index("s") * NCORES + lax.axis_index("c")


@functools.partial(
    pl.kernel,
    mesh=_mesh,
    out_type=jax.ShapeDtypeStruct((B, NI), jnp.float32),
    scratch_types=[
        pltpu.VMEM((ROWS, NSEEN), jnp.int32),
        pltpu.VMEM((ROWS, NCAND), jnp.int32),
        pltpu.VMEM((ROWS, NI), jnp.float32),
    ],
    compiler_params=_sc_params,
)
def _build_v(seen_hbm, item_hbm, v_hbm, seen_v, item_v, v_v):
    base = _wid() * ROWS
    pltpu.sync_copy(seen_hbm.at[pl.ds(base, ROWS)], seen_v)
    pltpu.sync_copy(item_hbm.at[pl.ds(base, ROWS)], item_v)

    zeros16 = jnp.zeros((16,), jnp.float32)
    ones16 = jnp.ones((16,), jnp.float32)

    def zbody(r, c):
        for j in range(NI // 16):
            v_v[r, pl.ds(j * 16, 16)] = zeros16
        v_v[r, pl.ds(NI - 16, 16)] = zeros16
        return c

    lax.fori_loop(0, ROWS, zbody, 0)

    lanes = lax.iota(jnp.int32, 16)
    row_groups = [lanes + g * 16 for g in range(ROWS // 16)]

    def sbody(k, c):
        s = k * 4
        for rows in row_groups:
            for j in range(4):
                col = jnp.full((16,), s + j, jnp.int32)
                vals = plsc.load_gather(seen_v, [rows, col])
                plsc.addupdate_scatter(v_v, [rows, vals], ones16)
        return c

    lax.fori_loop(0, NSEEN // 4, sbody, 0)

    def ibody(k, c):
        i = k * 4
        for rows in row_groups:
            for j in range(4):
                col = jnp.full((16,), i + j, jnp.int32)
                vals = plsc.load_gather(item_v, [rows, col])
                plsc.store_scatter(v_v, [rows, vals], zeros16)
        return c

    lax.fori_loop(0, NCAND // 4, ibody, 0)

    pltpu.sync_copy(v_v, v_hbm.at[pl.ds(base, ROWS)])


def _matmul_body(v_ref, w_ref, o_ref):
    o_ref[...] = lax.dot_general(
        v_ref[...], w_ref[...],
        (((1,), (1,)), ((), ())),
        preferred_element_type=jnp.float32,
    )


def _matmul(v, weights):
    return pl.pallas_call(
        _matmul_body,
        grid=(8,),
        in_specs=[
            pl.BlockSpec((B // 8, NI), lambda i: (i, 0)),
            pl.BlockSpec((NI, NI), lambda i: (0, 0)),
        ],
        out_specs=pl.BlockSpec((B // 8, NI), lambda i: (i, 0)),
        out_shape=jax.ShapeDtypeStruct((B, NI), jnp.float32),
    )(v, weights)


@functools.partial(
    pl.kernel,
    mesh=_mesh,
    out_type=jax.ShapeDtypeStruct((B, NCAND), jnp.float32),
    scratch_types=[
        pltpu.VMEM((ROWS, NI), jnp.float32),
        pltpu.VMEM((ROWS, NCAND), jnp.int32),
        pltpu.VMEM((ROWS, NCAND), jnp.float32),
    ],
    compiler_params=_sc_params,
)
def _gather_scores(scores_hbm, item_hbm, out_hbm, scores_v, item_v, out_v):
    base = _wid() * ROWS
    pltpu.sync_copy(scores_hbm.at[pl.ds(base, ROWS)], scores_v)
    pltpu.sync_copy(item_hbm.at[pl.ds(base, ROWS)], item_v)

    lanes = lax.iota(jnp.int32, 16)
    row_groups = [lanes + g * 16 for g in range(ROWS // 16)]

    def gbody(k, c):
        i = k * 4
        for rows in row_groups:
            for j in range(4):
                col = jnp.full((16,), i + j, jnp.int32)
                it = plsc.load_gather(item_v, [rows, col])
                vals = plsc.load_gather(scores_v, [rows, it])
                plsc.store_scatter(out_v, [rows, col], vals)
        return c

    lax.fori_loop(0, NCAND // 4, gbody, 0)

    pltpu.sync_copy(out_v, out_hbm.at[pl.ds(base, ROWS)])


def kernel(x, item, seen_items, weights):
    v = _build_v(seen_items, item)
    scores = _matmul(v, weights)
    return _gather_scores(scores, item)


# speedup vs baseline: 1.3045x; 1.3045x over previous
import functools

import jax
import jax.numpy as jnp
from jax import lax
from jax.experimental import pallas as pl
from jax.experimental.pallas import tpu as pltpu
from jax.experimental.pallas import tpu_sc as plsc

B = 1024
NI = 1000
NCAND = 100
NSEEN = 200

NCORES = 2
NSUB = 16
NW = NCORES * NSUB
ROWS = B // NW

_mesh = plsc.VectorSubcoreMesh(core_axis_name="c", subcore_axis_name="s")
_sc_params = pltpu.CompilerParams(needs_layout_passes=False)


def _wid():
    return lax.axis_index("s") * NCORES + lax.axis_index("c")


@functools.partial(
    pl.kernel,
    mesh=_mesh,
    out_type=jax.ShapeDtypeStruct((B, NI), jnp.float32),
    scratch_types=[
        pltpu.VMEM((ROWS, NSEEN), jnp.int32),
        pltpu.VMEM((ROWS, NCAND), jnp.int32),
        pltpu.VMEM((ROWS, NI), jnp.float32),
    ],
    compiler_params=_sc_params,
)
def _build_v(seen_hbm, item_hbm, v_hbm, seen_v, item_v, v_v):
    base = _wid() * ROWS
    pltpu.sync_copy(seen_hbm.at[pl.ds(base, ROWS)], seen_v)
    pltpu.sync_copy(item_hbm.at[pl.ds(base, ROWS)], item_v)

    zeros16 = jnp.zeros((16,), jnp.float32)
    ones16 = jnp.ones((16,), jnp.float32)

    @plsc.parallel_loop(0, ROWS, unroll=2)
    def _zero(r):
        for j in range(NI // 16):
            v_v[r, pl.ds(j * 16, 16)] = zeros16
        v_v[r, pl.ds(NI - 16, 16)] = zeros16

    lanes = lax.iota(jnp.int32, 16)
    row_groups = [lanes + g * 16 for g in range(ROWS // 16)]

    @plsc.parallel_loop(0, NSEEN, unroll=8)
    def _scatter_seen(s):
        col = jnp.full((16,), s, jnp.int32)
        for rows in row_groups:
            vals = plsc.load_gather(seen_v, [rows, col])
            plsc.addupdate_scatter(v_v, [rows, vals], ones16)

    @plsc.parallel_loop(0, NCAND, unroll=8)
    def _zero_items(i):
        col = jnp.full((16,), i, jnp.int32)
        for rows in row_groups:
            vals = plsc.load_gather(item_v, [rows, col])
            plsc.store_scatter(v_v, [rows, vals], zeros16)

    pltpu.sync_copy(v_v, v_hbm.at[pl.ds(base, ROWS)])


def _matmul_body(v_ref, w_ref, o_ref):
    o_ref[...] = lax.dot_general(
        v_ref[...], w_ref[...],
        (((1,), (1,)), ((), ())),
        preferred_element_type=jnp.float32,
    )


def _matmul(v, weights):
    return pl.pallas_call(
        _matmul_body,
        grid=(8,),
        in_specs=[
            pl.BlockSpec((B // 8, NI), lambda i: (i, 0)),
            pl.BlockSpec((NI, NI), lambda i: (0, 0)),
        ],
        out_specs=pl.BlockSpec((B // 8, NI), lambda i: (i, 0)),
        out_shape=jax.ShapeDtypeStruct((B, NI), jnp.float32),
    )(v, weights)


@functools.partial(
    pl.kernel,
    mesh=_mesh,
    out_type=jax.ShapeDtypeStruct((B, NCAND), jnp.float32),
    scratch_types=[
        pltpu.VMEM((ROWS, NI), jnp.float32),
        pltpu.VMEM((ROWS, NCAND), jnp.int32),
        pltpu.VMEM((ROWS, NCAND), jnp.float32),
    ],
    compiler_params=_sc_params,
)
def _gather_scores(scores_hbm, item_hbm, out_hbm, scores_v, item_v, out_v):
    base = _wid() * ROWS
    pltpu.sync_copy(scores_hbm.at[pl.ds(base, ROWS)], scores_v)
    pltpu.sync_copy(item_hbm.at[pl.ds(base, ROWS)], item_v)

    lanes = lax.iota(jnp.int32, 16)
    row_groups = [lanes + g * 16 for g in range(ROWS // 16)]

    @plsc.parallel_loop(0, NCAND, unroll=8)
    def _gather(i):
        col = jnp.full((16,), i, jnp.int32)
        for rows in row_groups:
            it = plsc.load_gather(item_v, [rows, col])
            vals = plsc.load_gather(scores_v, [rows, it])
            plsc.store_scatter(out_v, [rows, col], vals)

    pltpu.sync_copy(out_v, out_hbm.at[pl.ds(base, ROWS)])


def kernel(x, item, seen_items, weights):
    v = _build_v(seen_items, item)
    scores = _matmul(v, weights)
    return _gather_scores(scores, item)
